# baseline (speedup 1.0000x reference)
import functools

import jax
import jax.numpy as jnp
from jax.experimental import pallas as pl
from jax.experimental.pallas import tpu as pltpu

_B, _S, _D = 4, 4096, 2048
_IDX = 3
_BLK = 2048


def _tc_body(in_ref, out_ref):
    out_ref[0, :, :] = in_ref[0, :, _IDX:_IDX + 1]


def kernel(input_):
    grid = (_B, _S // _BLK)
    return pl.pallas_call(
        _tc_body,
        grid=grid,
        in_specs=[
            pl.BlockSpec((1, _BLK, 128), lambda b, s: (b, s, 0)),
        ],
        out_specs=pl.BlockSpec((1, _BLK, 1), lambda b, s: (b, s, 0)),
        out_shape=jax.ShapeDtypeStruct((_B, _S, 1), jnp.float32),
    )(input_)
